# Initial kernel scaffold; baseline (speedup 1.0000x reference)
import jax
import jax.numpy as jnp
from jax.experimental import pallas as pl


def kernel(x, edge_index, edge_attr, W1_l, b1, W1_r, W2_l, b2, W2_r):
    raise NotImplementedError("write your pallas kernel here")


# speedup vs baseline: 7.5702x; 7.5702x over previous
import functools

import jax
import jax.numpy as jnp
from jax import lax
from jax.experimental import pallas as pl
from jax.experimental.pallas import tpu as pltpu
from jax.experimental.pallas import tpu_sc as plsc

_N = 10000
_E = 320000
_F = 128
_C = 80
_NW = 32
_NCH = _E // (_NW * _C)
_NSUB = 16
_RPS = 624
_RPS_LAST = _N - 15 * _RPS
_BM = 1000


def _make_segsum(with_counts: bool):
    mesh = plsc.VectorSubcoreMesh(core_axis_name="c", subcore_axis_name="s")
    out_type = [jax.ShapeDtypeStruct((2, _N, _F), jnp.float32)]
    scratch = [
        pltpu.VMEM((_NCH, _C), jnp.int32),
        pltpu.VMEM((_NCH, _C), jnp.int32),
        pltpu.VMEM((_C, _F), jnp.float32),
        pltpu.VMEM_SHARED((_N, _F), jnp.float32),
        pltpu.SemaphoreType.DMA,
    ]
    if with_counts:
        out_type.append(jax.ShapeDtypeStruct((2, _N), jnp.float32))
        scratch += [
            pltpu.VMEM((_C,), jnp.float32),
            pltpu.VMEM_SHARED((_N,), jnp.float32),
        ]

    def body(y, src_i, dst_i, zrow, zc, ones, parts, counts,
             src_v, dst_v, rows_v, acc, sem, ones_v=None, cacc=None):
        c = lax.axis_index("c")
        s = lax.axis_index("s")
        wid = c * _NSUB + s
        @pl.when(s < 15)
        def _():
            pltpu.sync_copy(zrow.at[pl.ds(0, _RPS)],
                            acc.at[pl.ds(s * _RPS, _RPS)])

        @pl.when(s == 15)
        def _():
            pltpu.sync_copy(zrow, acc.at[pl.ds(15 * _RPS, _RPS_LAST)])
        if with_counts:
            @pl.when(s == 0)
            def _():
                pltpu.sync_copy(zc, cacc)
            pltpu.sync_copy(ones, ones_v)
        pltpu.sync_copy(src_i.at[wid], src_v)
        pltpu.sync_copy(dst_i.at[wid], dst_v)
        plsc.subcore_barrier()

        def chunk(j, carry):
            pltpu.async_copy(y.at[src_v.at[j]], rows_v, sem).wait()
            pltpu.sync_copy(rows_v, acc.at[dst_v.at[j]], add=True)
            if with_counts:
                pltpu.sync_copy(ones_v, cacc.at[dst_v.at[j]], add=True)
            return carry

        lax.fori_loop(0, _NCH, chunk, 0)
        plsc.subcore_barrier()

        @pl.when(s < 15)
        def _():
            pltpu.sync_copy(acc.at[pl.ds(s * _RPS, _RPS)],
                            parts.at[c, pl.ds(s * _RPS, _RPS)])

        @pl.when(s == 15)
        def _():
            pltpu.sync_copy(acc.at[pl.ds(15 * _RPS, _RPS_LAST)],
                            parts.at[c, pl.ds(15 * _RPS, _RPS_LAST)])
        if with_counts:
            @pl.when(s == 0)
            def _():
                pltpu.sync_copy(cacc, counts.at[c])

    if with_counts:
        def body_wc(y, src_i, dst_i, zrow, zc, ones, parts, counts,
                    src_v, dst_v, rows_v, acc, sem, ones_v, cacc):
            body(y, src_i, dst_i, zrow, zc, ones, parts, counts,
                 src_v, dst_v, rows_v, acc, sem, ones_v, cacc)
        fn = body_wc
    else:
        def body_nc(y, src_i, dst_i, zrow, zc, ones, parts,
                    src_v, dst_v, rows_v, acc, sem):
            body(y, src_i, dst_i, zrow, zc, ones, parts, None,
                 src_v, dst_v, rows_v, acc, sem)
        fn = body_nc

    return pl.kernel(fn, mesh=mesh, out_type=out_type, scratch_types=scratch)


_segsum_counts = _make_segsum(True)
_segsum = _make_segsum(False)


def _mm2_body(x_ref, wl_ref, wr_ref, b_ref, y_ref, r_ref):
    xb = x_ref[...]
    y_ref[...] = jnp.dot(xb, wl_ref[...], preferred_element_type=jnp.float32)
    r_ref[...] = (jnp.dot(xb, wr_ref[...], preferred_element_type=jnp.float32)
                  + b_ref[...])


_mm2 = pl.pallas_call(
    _mm2_body,
    grid=(_N // _BM,),
    in_specs=[
        pl.BlockSpec((_BM, _F), lambda i: (i, 0)),
        pl.BlockSpec((_F, _F), lambda i: (0, 0)),
        pl.BlockSpec((_F, _F), lambda i: (0, 0)),
        pl.BlockSpec((1, _F), lambda i: (0, 0)),
    ],
    out_specs=[pl.BlockSpec((_BM, _F), lambda i: (i, 0))] * 2,
    out_shape=[jax.ShapeDtypeStruct((_N, _F), jnp.float32)] * 2,
)


def _comb1_body(p_ref, c_ref, r1_ref, wl_ref, wr_ref, b_ref,
                y2_ref, r2_ref, inv_ref):
    p = p_ref[0] + p_ref[1]
    cnt = c_ref[0] + c_ref[1]
    inv = 1.0 / jnp.maximum(cnt, 1.0)
    h = jnp.maximum(p * inv + r1_ref[...], 0.0)
    y2_ref[...] = jnp.dot(h, wl_ref[...], preferred_element_type=jnp.float32)
    r2_ref[...] = (jnp.dot(h, wr_ref[...], preferred_element_type=jnp.float32)
                   + b_ref[...])
    inv_ref[...] = inv


_comb1 = pl.pallas_call(
    _comb1_body,
    grid=(_N // _BM,),
    in_specs=[
        pl.BlockSpec((2, _BM, _F), lambda i: (0, i, 0)),
        pl.BlockSpec((2, _BM, 1), lambda i: (0, i, 0)),
        pl.BlockSpec((_BM, _F), lambda i: (i, 0)),
        pl.BlockSpec((_F, _F), lambda i: (0, 0)),
        pl.BlockSpec((_F, _F), lambda i: (0, 0)),
        pl.BlockSpec((1, _F), lambda i: (0, 0)),
    ],
    out_specs=[
        pl.BlockSpec((_BM, _F), lambda i: (i, 0)),
        pl.BlockSpec((_BM, _F), lambda i: (i, 0)),
        pl.BlockSpec((_BM, 1), lambda i: (i, 0)),
    ],
    out_shape=[
        jax.ShapeDtypeStruct((_N, _F), jnp.float32),
        jax.ShapeDtypeStruct((_N, _F), jnp.float32),
        jax.ShapeDtypeStruct((_N, 1), jnp.float32),
    ],
)


def _comb2_body(p_ref, inv_ref, r2_ref, o_ref):
    p = p_ref[0] + p_ref[1]
    o_ref[...] = p * inv_ref[...] + r2_ref[...]


_comb2 = pl.pallas_call(
    _comb2_body,
    grid=(_N // _BM,),
    in_specs=[
        pl.BlockSpec((2, _BM, _F), lambda i: (0, i, 0)),
        pl.BlockSpec((_BM, 1), lambda i: (i, 0)),
        pl.BlockSpec((_BM, _F), lambda i: (i, 0)),
    ],
    out_specs=pl.BlockSpec((_BM, _F), lambda i: (i, 0)),
    out_shape=jax.ShapeDtypeStruct((_N, _F), jnp.float32),
)


def kernel(x, edge_index, edge_attr, W1_l, b1, W1_r, W2_l, b2, W2_r):
    del edge_attr
    src = edge_index[0].reshape(_NW, _NCH, _C).astype(jnp.int32)
    dst = edge_index[1].reshape(_NW, _NCH, _C).astype(jnp.int32)
    zrow = jnp.zeros((_RPS_LAST, _F), jnp.float32)
    zc = jnp.zeros((_N,), jnp.float32)
    ones = jnp.ones((_C,), jnp.float32)

    y1, r1 = _mm2(x, W1_l, W1_r, b1.reshape(1, _F))
    parts1, cnts = _segsum_counts(y1, src, dst, zrow, zc, ones)
    y2, r2, inv = _comb1(parts1, cnts.reshape(2, _N, 1), r1,
                         W2_l, W2_r, b2.reshape(1, _F))
    parts2, = _segsum(y2, src, dst, zrow, zc, ones)
    out = _comb2(parts2, inv, r2)
    return out


# speedup vs baseline: 11.2638x; 1.4879x over previous
import functools

import jax
import jax.numpy as jnp
from jax import lax
from jax.experimental import pallas as pl
from jax.experimental.pallas import tpu as pltpu
from jax.experimental.pallas import tpu_sc as plsc

_N = 10000
_E = 320000
_F = 128
_C = 50
_NW = 32
_NCH = _E // (_NW * _C)
_NSUB = 16
_RPS = 624
_RPS_LAST = _N - 15 * _RPS
_BM = 1000


_K = 4
_ST = 5
_SCH = _NCH // _ST
_NG = _SCH // _K


def _make_segsum(with_counts: bool):
    mesh = plsc.VectorSubcoreMesh(core_axis_name="c", subcore_axis_name="s")
    out_type = [jax.ShapeDtypeStruct((2, _N, _F), jnp.float32)]
    scratch = [
        pltpu.VMEM((_SCH, _C), jnp.int32),
        pltpu.VMEM((_SCH, _C), jnp.int32),
        [pltpu.VMEM((_C, _F), jnp.float32) for _ in range(_K)],
        pltpu.VMEM_SHARED((_N, _F), jnp.float32),
        [pltpu.SemaphoreType.DMA for _ in range(_K)],
        [pltpu.SemaphoreType.DMA for _ in range(_K)],
    ]
    if with_counts:
        out_type.append(jax.ShapeDtypeStruct((2, _N), jnp.float32))
        scratch += [
            pltpu.VMEM((_C,), jnp.float32),
            pltpu.VMEM_SHARED((_N,), jnp.float32),
            [pltpu.SemaphoreType.DMA for _ in range(_K)],
        ]

    def body(y, src_i, dst_i, zrow, zc, ones, parts, counts,
             src_v, dst_v, rows, acc, gsem, ssem,
             ones_v=None, cacc=None, csem=None):
        c = lax.axis_index("c")
        s = lax.axis_index("s")
        wid = c * _NSUB + s
        @pl.when(s < 15)
        def _():
            pltpu.sync_copy(zrow.at[pl.ds(0, _RPS)],
                            acc.at[pl.ds(s * _RPS, _RPS)])

        @pl.when(s == 15)
        def _():
            pltpu.sync_copy(zrow, acc.at[pl.ds(15 * _RPS, _RPS_LAST)])
        if with_counts:
            @pl.when(s == 0)
            def _():
                pltpu.sync_copy(zc, cacc)
            pltpu.sync_copy(ones, ones_v)
        plsc.subcore_barrier()

        def g_start(b, j):
            pltpu.async_copy(y.at[src_v.at[j]], rows[b], gsem[b])

        def g_wait(b):
            pltpu.make_async_copy(y.at[src_v.at[0]], rows[b], gsem[b]).wait()

        def s_start(b, j):
            pltpu.async_copy(rows[b], acc.at[dst_v.at[j]], ssem[b], add=True)
            if with_counts:
                pltpu.async_copy(ones_v, cacc.at[dst_v.at[j]], csem[b],
                                 add=True)

        def s_wait(b):
            pltpu.make_async_copy(rows[b], acc.at[dst_v.at[0]],
                                  ssem[b]).wait()
            if with_counts:
                pltpu.make_async_copy(ones_v, cacc.at[dst_v.at[0]],
                                      csem[b]).wait()

        def stage(t, carry):
            pltpu.sync_copy(src_i.at[wid, pl.ds(t * _SCH, _SCH)], src_v)
            pltpu.sync_copy(dst_i.at[wid, pl.ds(t * _SCH, _SCH)], dst_v)
            for b in range(_K):
                g_start(b, b)

            def round_(i, c2):
                for b in range(_K):
                    g_wait(b)
                    s_start(b, i * _K + b)
                for b in range(_K):
                    s_wait(b)

                    @pl.when(i + 1 < _NG)
                    def _():
                        g_start(b, (i + 1) * _K + b)
                return c2

            lax.fori_loop(0, _NG, round_, 0)
            return carry

        lax.fori_loop(0, _ST, stage, 0)
        plsc.subcore_barrier()

        @pl.when(s < 15)
        def _():
            pltpu.sync_copy(acc.at[pl.ds(s * _RPS, _RPS)],
                            parts.at[c, pl.ds(s * _RPS, _RPS)])

        @pl.when(s == 15)
        def _():
            pltpu.sync_copy(acc.at[pl.ds(15 * _RPS, _RPS_LAST)],
                            parts.at[c, pl.ds(15 * _RPS, _RPS_LAST)])
        if with_counts:
            @pl.when(s == 0)
            def _():
                pltpu.sync_copy(cacc, counts.at[c])

    if with_counts:
        def body_wc(y, src_i, dst_i, zrow, zc, ones, parts, counts,
                    src_v, dst_v, rows, acc, gsem, ssem, ones_v, cacc, csem):
            body(y, src_i, dst_i, zrow, zc, ones, parts, counts,
                 src_v, dst_v, rows, acc, gsem, ssem, ones_v, cacc, csem)
        fn = body_wc
    else:
        def body_nc(y, src_i, dst_i, zrow, zc, ones, parts,
                    src_v, dst_v, rows, acc, gsem, ssem):
            body(y, src_i, dst_i, zrow, zc, ones, parts, None,
                 src_v, dst_v, rows, acc, gsem, ssem)
        fn = body_nc

    return pl.kernel(fn, mesh=mesh, out_type=out_type, scratch_types=scratch)


_segsum_counts = _make_segsum(True)
_segsum = _make_segsum(False)


def _mm2_body(x_ref, wl_ref, wr_ref, b_ref, y_ref, r_ref):
    xb = x_ref[...]
    y_ref[...] = jnp.dot(xb, wl_ref[...], preferred_element_type=jnp.float32)
    r_ref[...] = (jnp.dot(xb, wr_ref[...], preferred_element_type=jnp.float32)
                  + b_ref[...])


_mm2 = pl.pallas_call(
    _mm2_body,
    grid=(_N // _BM,),
    in_specs=[
        pl.BlockSpec((_BM, _F), lambda i: (i, 0)),
        pl.BlockSpec((_F, _F), lambda i: (0, 0)),
        pl.BlockSpec((_F, _F), lambda i: (0, 0)),
        pl.BlockSpec((1, _F), lambda i: (0, 0)),
    ],
    out_specs=[pl.BlockSpec((_BM, _F), lambda i: (i, 0))] * 2,
    out_shape=[jax.ShapeDtypeStruct((_N, _F), jnp.float32)] * 2,
)


def _comb1_body(p_ref, c_ref, r1_ref, wl_ref, wr_ref, b_ref,
                y2_ref, r2_ref, inv_ref):
    p = p_ref[0] + p_ref[1]
    cnt = c_ref[0] + c_ref[1]
    inv = 1.0 / jnp.maximum(cnt, 1.0)
    h = jnp.maximum(p * inv + r1_ref[...], 0.0)
    y2_ref[...] = jnp.dot(h, wl_ref[...], preferred_element_type=jnp.float32)
    r2_ref[...] = (jnp.dot(h, wr_ref[...], preferred_element_type=jnp.float32)
                   + b_ref[...])
    inv_ref[...] = inv


_comb1 = pl.pallas_call(
    _comb1_body,
    grid=(_N // _BM,),
    in_specs=[
        pl.BlockSpec((2, _BM, _F), lambda i: (0, i, 0)),
        pl.BlockSpec((2, _BM, 1), lambda i: (0, i, 0)),
        pl.BlockSpec((_BM, _F), lambda i: (i, 0)),
        pl.BlockSpec((_F, _F), lambda i: (0, 0)),
        pl.BlockSpec((_F, _F), lambda i: (0, 0)),
        pl.BlockSpec((1, _F), lambda i: (0, 0)),
    ],
    out_specs=[
        pl.BlockSpec((_BM, _F), lambda i: (i, 0)),
        pl.BlockSpec((_BM, _F), lambda i: (i, 0)),
        pl.BlockSpec((_BM, 1), lambda i: (i, 0)),
    ],
    out_shape=[
        jax.ShapeDtypeStruct((_N, _F), jnp.float32),
        jax.ShapeDtypeStruct((_N, _F), jnp.float32),
        jax.ShapeDtypeStruct((_N, 1), jnp.float32),
    ],
)


def _comb2_body(p_ref, inv_ref, r2_ref, o_ref):
    p = p_ref[0] + p_ref[1]
    o_ref[...] = p * inv_ref[...] + r2_ref[...]


_comb2 = pl.pallas_call(
    _comb2_body,
    grid=(_N // _BM,),
    in_specs=[
        pl.BlockSpec((2, _BM, _F), lambda i: (0, i, 0)),
        pl.BlockSpec((_BM, 1), lambda i: (i, 0)),
        pl.BlockSpec((_BM, _F), lambda i: (i, 0)),
    ],
    out_specs=pl.BlockSpec((_BM, _F), lambda i: (i, 0)),
    out_shape=jax.ShapeDtypeStruct((_N, _F), jnp.float32),
)


def kernel(x, edge_index, edge_attr, W1_l, b1, W1_r, W2_l, b2, W2_r):
    del edge_attr
    src = edge_index[0].reshape(_NW, _NCH, _C).astype(jnp.int32)
    dst = edge_index[1].reshape(_NW, _NCH, _C).astype(jnp.int32)
    zrow = jnp.zeros((_RPS_LAST, _F), jnp.float32)
    zc = jnp.zeros((_N,), jnp.float32)
    ones = jnp.ones((_C,), jnp.float32)

    y1, r1 = _mm2(x, W1_l, W1_r, b1.reshape(1, _F))
    parts1, cnts = _segsum_counts(y1, src, dst, zrow, zc, ones)
    y2, r2, inv = _comb1(parts1, cnts.reshape(2, _N, 1), r1,
                         W2_l, W2_r, b2.reshape(1, _F))
    parts2, = _segsum(y2, src, dst, zrow, zc, ones)
    out = _comb2(parts2, inv, r2)
    return out


# speedup vs baseline: 11.5980x; 1.0297x over previous
import functools

import jax
import jax.numpy as jnp
from jax import lax
from jax.experimental import pallas as pl
from jax.experimental.pallas import tpu as pltpu
from jax.experimental.pallas import tpu_sc as plsc

_N = 10000
_E = 320000
_F = 128
_C = 50
_NW = 32
_NCH = _E // (_NW * _C)
_NSUB = 16
_RPS = 624
_RPS_LAST = _N - 15 * _RPS
_BM = 1000


_K = 2
_ST = 5
_SCH = _NCH // _ST
_NR = _SCH // _K
_NPAIR = _NR // 2


def _make_segsum(with_counts: bool):
    mesh = plsc.VectorSubcoreMesh(core_axis_name="c", subcore_axis_name="s")
    out_type = [jax.ShapeDtypeStruct((2, _N, _F), jnp.float32)]
    scratch = [
        pltpu.VMEM((_SCH, _C), jnp.int32),
        pltpu.VMEM((_SCH, _C), jnp.int32),
        [[pltpu.VMEM((_C, _F), jnp.float32) for _ in range(_K)]
         for _ in range(2)],
        pltpu.VMEM_SHARED((_N, _F), jnp.float32),
        [[pltpu.SemaphoreType.DMA for _ in range(_K)] for _ in range(2)],
        [[pltpu.SemaphoreType.DMA for _ in range(_K)] for _ in range(2)],
    ]
    if with_counts:
        out_type.append(jax.ShapeDtypeStruct((2, _N), jnp.float32))
        scratch += [
            pltpu.VMEM((_C,), jnp.float32),
            pltpu.VMEM_SHARED((_N,), jnp.float32),
            [[pltpu.SemaphoreType.DMA for _ in range(_K)] for _ in range(2)],
        ]

    def body(y, src_i, dst_i, zrow, zc, ones, parts, counts,
             src_v, dst_v, rows, acc, gsem, ssem,
             ones_v=None, cacc=None, csem=None):
        c = lax.axis_index("c")
        s = lax.axis_index("s")
        wid = c * _NSUB + s
        @pl.when(s < 15)
        def _():
            pltpu.sync_copy(zrow.at[pl.ds(0, _RPS)],
                            acc.at[pl.ds(s * _RPS, _RPS)])

        @pl.when(s == 15)
        def _():
            pltpu.sync_copy(zrow, acc.at[pl.ds(15 * _RPS, _RPS_LAST)])
        if with_counts:
            @pl.when(s == 0)
            def _():
                pltpu.sync_copy(zc, cacc)
            pltpu.sync_copy(ones, ones_v)
        plsc.subcore_barrier()

        def g_start(st, b, j):
            pltpu.async_copy(y.at[src_v.at[j]], rows[st][b], gsem[st][b])

        def g_wait(st, b):
            pltpu.make_async_copy(y.at[src_v.at[0]], rows[st][b],
                                  gsem[st][b]).wait()

        def s_start(st, b, j):
            pltpu.async_copy(rows[st][b], acc.at[dst_v.at[j]], ssem[st][b],
                             add=True)
            if with_counts:
                pltpu.async_copy(ones_v, cacc.at[dst_v.at[j]], csem[st][b],
                                 add=True)

        def s_wait(st, b):
            pltpu.make_async_copy(rows[st][b], acc.at[dst_v.at[0]],
                                  ssem[st][b]).wait()
            if with_counts:
                pltpu.make_async_copy(ones_v, cacc.at[dst_v.at[0]],
                                      csem[st][b]).wait()

        def stage(t, carry):
            pltpu.sync_copy(src_i.at[wid, pl.ds(t * _SCH, _SCH)], src_v)
            pltpu.sync_copy(dst_i.at[wid, pl.ds(t * _SCH, _SCH)], dst_v)
            for st in range(2):
                for b in range(_K):
                    g_start(st, b, st * _K + b)

            def pair(p, c2):
                for st in range(2):
                    base = (2 * p + st) * _K
                    for b in range(_K):
                        g_wait(st, b)
                        s_start(st, b, base + b)
                    for b in range(_K):
                        @pl.when(p > 0)
                        def _():
                            s_wait(st, b)

                        @pl.when(p + 1 < _NPAIR)
                        def _():
                            g_start(st, b, base + 2 * _K + b)
                return c2

            lax.fori_loop(0, _NPAIR, pair, 0)
            for st in range(2):
                for b in range(_K):
                    s_wait(st, b)
            return carry

        lax.fori_loop(0, _ST, stage, 0)
        plsc.subcore_barrier()

        @pl.when(s < 15)
        def _():
            pltpu.sync_copy(acc.at[pl.ds(s * _RPS, _RPS)],
                            parts.at[c, pl.ds(s * _RPS, _RPS)])

        @pl.when(s == 15)
        def _():
            pltpu.sync_copy(acc.at[pl.ds(15 * _RPS, _RPS_LAST)],
                            parts.at[c, pl.ds(15 * _RPS, _RPS_LAST)])
        if with_counts:
            @pl.when(s == 0)
            def _():
                pltpu.sync_copy(cacc, counts.at[c])

    if with_counts:
        def body_wc(y, src_i, dst_i, zrow, zc, ones, parts, counts,
                    src_v, dst_v, rows, acc, gsem, ssem, ones_v, cacc, csem):
            body(y, src_i, dst_i, zrow, zc, ones, parts, counts,
                 src_v, dst_v, rows, acc, gsem, ssem, ones_v, cacc, csem)
        fn = body_wc
    else:
        def body_nc(y, src_i, dst_i, zrow, zc, ones, parts,
                    src_v, dst_v, rows, acc, gsem, ssem):
            body(y, src_i, dst_i, zrow, zc, ones, parts, None,
                 src_v, dst_v, rows, acc, gsem, ssem)
        fn = body_nc

    return pl.kernel(fn, mesh=mesh, out_type=out_type, scratch_types=scratch)


_segsum_counts = _make_segsum(True)
_segsum = _make_segsum(False)


def _mm2_body(x_ref, wl_ref, wr_ref, b_ref, y_ref, r_ref):
    xb = x_ref[...]
    y_ref[...] = jnp.dot(xb, wl_ref[...], preferred_element_type=jnp.float32)
    r_ref[...] = (jnp.dot(xb, wr_ref[...], preferred_element_type=jnp.float32)
                  + b_ref[...])


_mm2 = pl.pallas_call(
    _mm2_body,
    grid=(_N // _BM,),
    in_specs=[
        pl.BlockSpec((_BM, _F), lambda i: (i, 0)),
        pl.BlockSpec((_F, _F), lambda i: (0, 0)),
        pl.BlockSpec((_F, _F), lambda i: (0, 0)),
        pl.BlockSpec((1, _F), lambda i: (0, 0)),
    ],
    out_specs=[pl.BlockSpec((_BM, _F), lambda i: (i, 0))] * 2,
    out_shape=[jax.ShapeDtypeStruct((_N, _F), jnp.float32)] * 2,
)


def _comb1_body(p_ref, c_ref, r1_ref, wl_ref, wr_ref, b_ref,
                y2_ref, r2_ref, inv_ref):
    p = p_ref[0] + p_ref[1]
    cnt = c_ref[0] + c_ref[1]
    inv = 1.0 / jnp.maximum(cnt, 1.0)
    h = jnp.maximum(p * inv + r1_ref[...], 0.0)
    y2_ref[...] = jnp.dot(h, wl_ref[...], preferred_element_type=jnp.float32)
    r2_ref[...] = (jnp.dot(h, wr_ref[...], preferred_element_type=jnp.float32)
                   + b_ref[...])
    inv_ref[...] = inv


_comb1 = pl.pallas_call(
    _comb1_body,
    grid=(_N // _BM,),
    in_specs=[
        pl.BlockSpec((2, _BM, _F), lambda i: (0, i, 0)),
        pl.BlockSpec((2, _BM, 1), lambda i: (0, i, 0)),
        pl.BlockSpec((_BM, _F), lambda i: (i, 0)),
        pl.BlockSpec((_F, _F), lambda i: (0, 0)),
        pl.BlockSpec((_F, _F), lambda i: (0, 0)),
        pl.BlockSpec((1, _F), lambda i: (0, 0)),
    ],
    out_specs=[
        pl.BlockSpec((_BM, _F), lambda i: (i, 0)),
        pl.BlockSpec((_BM, _F), lambda i: (i, 0)),
        pl.BlockSpec((_BM, 1), lambda i: (i, 0)),
    ],
    out_shape=[
        jax.ShapeDtypeStruct((_N, _F), jnp.float32),
        jax.ShapeDtypeStruct((_N, _F), jnp.float32),
        jax.ShapeDtypeStruct((_N, 1), jnp.float32),
    ],
)


def _comb2_body(p_ref, inv_ref, r2_ref, o_ref):
    p = p_ref[0] + p_ref[1]
    o_ref[...] = p * inv_ref[...] + r2_ref[...]


_comb2 = pl.pallas_call(
    _comb2_body,
    grid=(_N // _BM,),
    in_specs=[
        pl.BlockSpec((2, _BM, _F), lambda i: (0, i, 0)),
        pl.BlockSpec((_BM, 1), lambda i: (i, 0)),
        pl.BlockSpec((_BM, _F), lambda i: (i, 0)),
    ],
    out_specs=pl.BlockSpec((_BM, _F), lambda i: (i, 0)),
    out_shape=jax.ShapeDtypeStruct((_N, _F), jnp.float32),
)


def kernel(x, edge_index, edge_attr, W1_l, b1, W1_r, W2_l, b2, W2_r):
    del edge_attr
    src = edge_index[0].reshape(_NW, _NCH, _C).astype(jnp.int32)
    dst = edge_index[1].reshape(_NW, _NCH, _C).astype(jnp.int32)
    zrow = jnp.zeros((_RPS_LAST, _F), jnp.float32)
    zc = jnp.zeros((_N,), jnp.float32)
    ones = jnp.ones((_C,), jnp.float32)

    y1, r1 = _mm2(x, W1_l, W1_r, b1.reshape(1, _F))
    parts1, cnts = _segsum_counts(y1, src, dst, zrow, zc, ones)
    y2, r2, inv = _comb1(parts1, cnts.reshape(2, _N, 1), r1,
                         W2_l, W2_r, b2.reshape(1, _F))
    parts2, = _segsum(y2, src, dst, zrow, zc, ones)
    out = _comb2(parts2, inv, r2)
    return out
